# baseline (speedup 1.0000x reference)
import functools

import jax
import jax.numpy as jnp
from jax import lax
from jax.experimental import pallas as pl
from jax.experimental.pallas import tpu as pltpu
from jax.experimental.pallas import tpu_sc as plsc

HW = 4096
CH = 32
BZ = 4
QCHUNK = 512
NCHUNK = HW // QCHUNK
TBL_ROWS = 2 * BZ * HW


def _argmax_body(former_ref, latter_ref, flagrow_ref, flagcol_ref,
                 out_ref, table_ref):
    b = pl.program_id(0)
    i = pl.program_id(1)
    lat = latter_ref[0, 0]
    norm = jnp.sqrt(jnp.sum(lat * lat, axis=0, keepdims=True)) + 1e-8
    lat_n = lat / norm

    neg = flagrow_ref[...].astype(jnp.float32) * jnp.float32(-1e30)
    lat_aug = jnp.concatenate([lat_n, neg], axis=0)
    f = former_ref[0, 0]
    f_aug = jnp.concatenate(
        [f, jnp.ones((1, QCHUNK), jnp.float32)], axis=0)
    cos = lax.dot_general(
        f_aug, lat_aug, (((0,), (0,)), ((), ())),
        preferred_element_type=jnp.float32)

    maxv = jnp.max(cos, axis=1, keepdims=True)
    rev = (jnp.float32(HW) -
           lax.broadcasted_iota(jnp.int32, (1, HW), 1).astype(jnp.float32))
    idx = (jnp.float32(HW) -
           jnp.max(jnp.where(cos == maxv, rev, jnp.float32(0.0)),
                   axis=1, keepdims=True)).astype(jnp.int32)

    hole_q = flagcol_ref[...] > 0
    riota = lax.broadcasted_iota(jnp.int32, (QCHUNK, 1), 0)
    data_idx = ((idx & jnp.int32(-512)) * 2 + ((idx & 127) * 4)
                + ((idx // 128) & 3) + b * 2 * HW)
    pad_idx = b * 2 * HW + i * 2 * QCHUNK + QCHUNK + riota
    out_ref[0] = jnp.where(hole_q, data_idx, pad_idx)

    lsl = latter_ref[0, 0, :, pl.ds(i * QCHUNK, QCHUNK)]
    d128 = jnp.concatenate(
        [lsl[:, a * 128:(a + 1) * 128].T for a in range(4)], axis=1)
    table_ref[...] = jnp.concatenate(
        [d128, jnp.zeros_like(d128)], axis=0)


def _nn_indices(input3d, flagrow, flagcol):
    inspect = input3d.reshape(BZ, 2, CH, HW)
    return pl.pallas_call(
        _argmax_body,
        grid=(BZ, NCHUNK),
        in_specs=[
            pl.BlockSpec((1, 1, CH, QCHUNK), lambda b, i: (b, 0, 0, i)),
            pl.BlockSpec((1, 1, CH, HW), lambda b, i: (b, 1, 0, 0)),
            pl.BlockSpec((1, HW), lambda b, i: (0, 0)),
            pl.BlockSpec((QCHUNK, 1), lambda b, i: (i, 0)),
        ],
        out_specs=[
            pl.BlockSpec((1, QCHUNK, 1), lambda b, i: (b, i, 0)),
            pl.BlockSpec((2 * QCHUNK * CH // 128, 128),
                         lambda b, i: (b * NCHUNK + i, 0)),
        ],
        out_shape=[
            jax.ShapeDtypeStruct((BZ, HW, 1), jnp.int32),
            jax.ShapeDtypeStruct((TBL_ROWS * CH // 128, 128), jnp.float32),
        ],
    )(inspect, inspect, flagrow, flagcol)


_NC, _NS = 2, 16
_NW = _NC * _NS
_ROWS_PER_W = BZ * HW // _NW
_IDX_LANES = 128
_GATHERS = _ROWS_PER_W // _IDX_LANES


def _sc_gather_body(table_hbm, idx_hbm, out_hbm, idx_v, rows_v, sem):
    wid = lax.axis_index("s") * _NC + lax.axis_index("c")
    pltpu.sync_copy(idx_hbm.at[pl.ds(wid * _GATHERS, _GATHERS)], idx_v)
    copies = [
        pltpu.async_copy(table_hbm.at[idx_v.at[j]],
                         rows_v.at[pl.ds(j * _IDX_LANES, _IDX_LANES)], sem)
        for j in range(_GATHERS)
    ]
    for c in copies:
        c.wait()
    for a in range(_GATHERS):
        pltpu.sync_copy(
            rows_v.at[pl.ds(a * _IDX_LANES, _IDX_LANES)],
            out_hbm.at[pl.ds(wid * _IDX_LANES, _IDX_LANES),
                       pl.ds(a * CH, CH)])


@functools.cache
def _sc_gather_kernel():
    return pl.kernel(
        _sc_gather_body,
        out_type=jax.ShapeDtypeStruct((BZ * HW // 4, 128), jnp.float32),
        mesh=plsc.VectorSubcoreMesh(core_axis_name="c", subcore_axis_name="s"),
        scratch_types=[
            pltpu.VMEM((_GATHERS, _IDX_LANES), jnp.int32),
            pltpu.VMEM((_GATHERS * _IDX_LANES, CH), jnp.float32),
            pltpu.SemaphoreType.DMA,
        ],
        compiler_params=pltpu.CompilerParams(use_tc_tiling_on_sc=False),
    )


def _assemble_body(in_ref, shift_ref, out_ref):
    xin = in_ref[0]
    s = shift_ref[...]
    y = jnp.concatenate(
        [s[:, a * CH:(a + 1) * CH].T for a in range(4)], axis=1)
    out_ref[0] = jnp.concatenate([xin, y], axis=0)


def _assemble(input3d, shifted128):
    return pl.pallas_call(
        _assemble_body,
        grid=(BZ, NCHUNK),
        in_specs=[
            pl.BlockSpec((1, 2 * CH, QCHUNK), lambda b, i: (b, 0, i)),
            pl.BlockSpec((QCHUNK // 4, 128), lambda b, i: (b * NCHUNK + i, 0)),
        ],
        out_specs=pl.BlockSpec((1, 3 * CH, QCHUNK), lambda b, i: (b, 0, i)),
        out_shape=jax.ShapeDtypeStruct((BZ, 3 * CH, HW), jnp.float32),
    )(input3d, shifted128)


@jax.jit
def kernel(input, mask):
    bz, c, h, w = input.shape
    ch = c // 2
    input3d = input.reshape(bz, c, h * w)

    flag = (mask.reshape(1, h * w) >= 1).astype(jnp.int32)
    gidx, table128 = _nn_indices(input3d, flag, flag.reshape(h * w, 1))

    shifted = _sc_gather_kernel()(table128.reshape(TBL_ROWS, ch),
                                  gidx.reshape(-1, _IDX_LANES))
    out = _assemble(input3d, shifted)
    return out.reshape(bz, 3 * ch, h, w)


# speedup vs baseline: 1.0885x; 1.0885x over previous
import functools

import jax
import jax.numpy as jnp
from jax import lax
from jax.experimental import pallas as pl
from jax.experimental.pallas import tpu as pltpu
from jax.experimental.pallas import tpu_sc as plsc

HW = 4096
CH = 32
BZ = 4
QCHUNK = 1024
NCHUNK = HW // QCHUNK
NBAND = QCHUNK // 128
TBL_ROWS = 2 * BZ * HW


def _argmax_body(former_ref, latter_ref, flagrow_ref, flagcol_ref,
                 out_ref, table_ref):
    b = pl.program_id(0)
    i = pl.program_id(1)
    lat = latter_ref[0, 0]
    norm = jnp.sqrt(jnp.sum(lat * lat, axis=0, keepdims=True)) + 1e-8
    lat_n = lat / norm

    neg = flagrow_ref[...].astype(jnp.float32) * jnp.float32(-1e30)
    lat_aug = jnp.concatenate([lat_n, neg], axis=0)
    f = former_ref[0, 0]
    f_aug = jnp.concatenate(
        [f, jnp.ones((1, QCHUNK), jnp.float32)], axis=0)
    cos = lax.dot_general(
        f_aug, lat_aug, (((0,), (0,)), ((), ())),
        preferred_element_type=jnp.float32)

    maxv = jnp.max(cos, axis=1, keepdims=True)
    rev = (jnp.float32(HW) -
           lax.broadcasted_iota(jnp.int32, (1, HW), 1).astype(jnp.float32))
    idx = (jnp.float32(HW) -
           jnp.max(jnp.where(cos == maxv, rev, jnp.float32(0.0)),
                   axis=1, keepdims=True)).astype(jnp.int32)

    hole_q = flagcol_ref[...] > 0
    riota = lax.broadcasted_iota(jnp.int32, (QCHUNK, 1), 0)
    data_idx = ((idx & jnp.int32(-QCHUNK)) * 2
                + ((idx // 512) & (NBAND // 4 - 1)) * 512
                + ((idx & 127) * 4)
                + ((idx // 128) & 3) + b * 2 * HW)
    pad_idx = b * 2 * HW + i * 2 * QCHUNK + QCHUNK + riota
    out_ref[0] = jnp.where(hole_q, data_idx, pad_idx)

    lsl = latter_ref[0, 0, :, pl.ds(i * QCHUNK, QCHUNK)]
    bands = [lsl[:, a * 128:(a + 1) * 128].T for a in range(NBAND)]
    groups = [jnp.concatenate(bands[g * 4:(g + 1) * 4], axis=1)
              for g in range(NBAND // 4)]
    data = jnp.concatenate(groups, axis=0)
    table_ref[...] = jnp.concatenate(
        [data, jnp.zeros_like(data)], axis=0)


def _nn_indices(input3d, flagrow, flagcol):
    inspect = input3d.reshape(BZ, 2, CH, HW)
    return pl.pallas_call(
        _argmax_body,
        grid=(BZ, NCHUNK),
        in_specs=[
            pl.BlockSpec((1, 1, CH, QCHUNK), lambda b, i: (b, 0, 0, i)),
            pl.BlockSpec((1, 1, CH, HW), lambda b, i: (b, 1, 0, 0)),
            pl.BlockSpec((1, HW), lambda b, i: (0, 0)),
            pl.BlockSpec((QCHUNK, 1), lambda b, i: (i, 0)),
        ],
        out_specs=[
            pl.BlockSpec((1, QCHUNK, 1), lambda b, i: (b, i, 0)),
            pl.BlockSpec((2 * QCHUNK * CH // 128, 128),
                         lambda b, i: (b * NCHUNK + i, 0)),
        ],
        out_shape=[
            jax.ShapeDtypeStruct((BZ, HW, 1), jnp.int32),
            jax.ShapeDtypeStruct((TBL_ROWS * CH // 128, 128), jnp.float32),
        ],
    )(inspect, inspect, flagrow, flagcol)


_NC, _NS = 2, 16
_NW = _NC * _NS
_ROWS_PER_W = BZ * HW // _NW
_IDX_LANES = 128
_GATHERS = _ROWS_PER_W // _IDX_LANES


def _sc_gather_body(table_hbm, idx_hbm, out_hbm, idx_v, rows_v, sem):
    wid = lax.axis_index("s") * _NC + lax.axis_index("c")
    pltpu.sync_copy(idx_hbm.at[pl.ds(wid * _GATHERS, _GATHERS)], idx_v)
    copies = [
        pltpu.async_copy(table_hbm.at[idx_v.at[j]],
                         rows_v.at[pl.ds(j * _IDX_LANES, _IDX_LANES)], sem)
        for j in range(_GATHERS)
    ]
    for c in copies:
        c.wait()
    pltpu.sync_copy(rows_v,
                    out_hbm.at[pl.ds(wid * _ROWS_PER_W, _ROWS_PER_W)])


@functools.cache
def _sc_gather_kernel():
    return pl.kernel(
        _sc_gather_body,
        out_type=jax.ShapeDtypeStruct((BZ * HW, CH), jnp.float32),
        mesh=plsc.VectorSubcoreMesh(core_axis_name="c", subcore_axis_name="s"),
        scratch_types=[
            pltpu.VMEM((_GATHERS, _IDX_LANES), jnp.int32),
            pltpu.VMEM((_GATHERS * _IDX_LANES, CH), jnp.float32),
            pltpu.SemaphoreType.DMA,
        ],
        compiler_params=pltpu.CompilerParams(use_tc_tiling_on_sc=False),
    )


@jax.jit
def kernel(input, mask):
    bz, c, h, w = input.shape
    ch = c // 2
    input3d = input.reshape(bz, c, h * w)

    flag = (mask.reshape(1, h * w) >= 1).astype(jnp.int32)
    gidx, table128 = _nn_indices(input3d, flag, flag.reshape(h * w, 1))

    shifted = _sc_gather_kernel()(table128.reshape(TBL_ROWS, ch),
                                  gidx.reshape(-1, _IDX_LANES))
    shift = shifted.reshape(bz, h * w, ch).transpose(0, 2, 1)
    return jnp.concatenate([input, shift.reshape(bz, ch, h, w)], axis=1)


# speedup vs baseline: 1.1054x; 1.0155x over previous
import functools

import jax
import jax.numpy as jnp
from jax import lax
from jax.experimental import pallas as pl
from jax.experimental.pallas import tpu as pltpu
from jax.experimental.pallas import tpu_sc as plsc

HW = 4096
CH = 32
BZ = 4
QCHUNK = 2048
NCHUNK = HW // QCHUNK
NBAND = QCHUNK // 128
TBL_ROWS = 2 * BZ * HW


def _argmax_body(former_ref, latter_ref, flagrow_ref, flagcol_ref,
                 out_ref, table_ref):
    b = pl.program_id(0)
    i = pl.program_id(1)
    lat = latter_ref[0, 0]
    norm = jnp.sqrt(jnp.sum(lat * lat, axis=0, keepdims=True)) + 1e-8
    lat_n = lat / norm

    neg = flagrow_ref[...].astype(jnp.float32) * jnp.float32(-1e30)
    lat_aug = jnp.concatenate([lat_n, neg], axis=0)
    f = former_ref[0, 0]
    f_aug = jnp.concatenate(
        [f, jnp.ones((1, QCHUNK), jnp.float32)], axis=0)
    cos = lax.dot_general(
        f_aug, lat_aug, (((0,), (0,)), ((), ())),
        preferred_element_type=jnp.float32)

    maxv = jnp.max(cos, axis=1, keepdims=True)
    rev = (jnp.float32(HW) -
           lax.broadcasted_iota(jnp.int32, (1, HW), 1).astype(jnp.float32))
    idx = (jnp.float32(HW) -
           jnp.max(jnp.where(cos == maxv, rev, jnp.float32(0.0)),
                   axis=1, keepdims=True)).astype(jnp.int32)

    hole_q = flagcol_ref[...] > 0
    riota = lax.broadcasted_iota(jnp.int32, (QCHUNK, 1), 0)
    data_idx = ((idx & jnp.int32(-QCHUNK)) * 2
                + ((idx // 512) & (NBAND // 4 - 1)) * 512
                + ((idx & 127) * 4)
                + ((idx // 128) & 3) + b * 2 * HW)
    pad_idx = b * 2 * HW + i * 2 * QCHUNK + QCHUNK + riota
    out_ref[0] = jnp.where(hole_q, data_idx, pad_idx)

    lsl = latter_ref[0, 0, :, pl.ds(i * QCHUNK, QCHUNK)]
    bands = [lsl[:, a * 128:(a + 1) * 128].T for a in range(NBAND)]
    groups = [jnp.concatenate(bands[g * 4:(g + 1) * 4], axis=1)
              for g in range(NBAND // 4)]
    data = jnp.concatenate(groups, axis=0)
    table_ref[...] = jnp.concatenate(
        [data, jnp.zeros_like(data)], axis=0)


def _nn_indices(input3d, flagrow, flagcol):
    inspect = input3d.reshape(BZ, 2, CH, HW)
    return pl.pallas_call(
        _argmax_body,
        grid=(BZ, NCHUNK),
        in_specs=[
            pl.BlockSpec((1, 1, CH, QCHUNK), lambda b, i: (b, 0, 0, i)),
            pl.BlockSpec((1, 1, CH, HW), lambda b, i: (b, 1, 0, 0)),
            pl.BlockSpec((1, HW), lambda b, i: (0, 0)),
            pl.BlockSpec((QCHUNK, 1), lambda b, i: (i, 0)),
        ],
        out_specs=[
            pl.BlockSpec((1, QCHUNK, 1), lambda b, i: (b, i, 0)),
            pl.BlockSpec((2 * QCHUNK * CH // 128, 128),
                         lambda b, i: (b * NCHUNK + i, 0)),
        ],
        out_shape=[
            jax.ShapeDtypeStruct((BZ, HW, 1), jnp.int32),
            jax.ShapeDtypeStruct((TBL_ROWS * CH // 128, 128), jnp.float32),
        ],
    )(inspect, inspect, flagrow, flagcol)


_NC, _NS = 2, 16
_NW = _NC * _NS
_ROWS_PER_W = BZ * HW // _NW
_IDX_LANES = 128
_GATHERS = _ROWS_PER_W // _IDX_LANES


def _sc_gather_body(table_hbm, idx_hbm, out_hbm, idx_v, rows_v, sem):
    wid = lax.axis_index("s") * _NC + lax.axis_index("c")
    pltpu.sync_copy(idx_hbm.at[pl.ds(wid * _GATHERS, _GATHERS)], idx_v)
    copies = [
        pltpu.async_copy(table_hbm.at[idx_v.at[j]],
                         rows_v.at[pl.ds(j * _IDX_LANES, _IDX_LANES)], sem)
        for j in range(_GATHERS)
    ]
    for c in copies:
        c.wait()
    pltpu.sync_copy(rows_v,
                    out_hbm.at[pl.ds(wid * _ROWS_PER_W, _ROWS_PER_W)])


@functools.cache
def _sc_gather_kernel():
    return pl.kernel(
        _sc_gather_body,
        out_type=jax.ShapeDtypeStruct((BZ * HW, CH), jnp.float32),
        mesh=plsc.VectorSubcoreMesh(core_axis_name="c", subcore_axis_name="s"),
        scratch_types=[
            pltpu.VMEM((_GATHERS, _IDX_LANES), jnp.int32),
            pltpu.VMEM((_GATHERS * _IDX_LANES, CH), jnp.float32),
            pltpu.SemaphoreType.DMA,
        ],
        compiler_params=pltpu.CompilerParams(use_tc_tiling_on_sc=False),
    )


@jax.jit
def kernel(input, mask):
    bz, c, h, w = input.shape
    ch = c // 2
    input3d = input.reshape(bz, c, h * w)

    flag = (mask.reshape(1, h * w) >= 1).astype(jnp.int32)
    gidx, table128 = _nn_indices(input3d, flag, flag.reshape(h * w, 1))

    shifted = _sc_gather_kernel()(table128.reshape(TBL_ROWS, ch),
                                  gidx.reshape(-1, _IDX_LANES))
    shift = shifted.reshape(bz, h * w, ch).transpose(0, 2, 1)
    return jnp.concatenate([input, shift.reshape(bz, ch, h, w)], axis=1)
